# Initial kernel scaffold; baseline (speedup 1.0000x reference)
import jax
import jax.numpy as jnp
from jax.experimental import pallas as pl


def kernel(x_user, x_item, W_u2i, W_i2u, src_idx_u2i, dst_idx_u2i, src_idx_i2u, dst_idx_i2u):
    raise NotImplementedError("write your pallas kernel here")


# speedup vs baseline: 4.8864x; 4.8864x over previous
import functools

import jax
import jax.numpy as jnp
from jax import lax
from jax.experimental import pallas as pl
from jax.experimental.pallas import tpu as pltpu
from jax.experimental.pallas import tpu_sc as plsc

N_NODE = 10000
E_EDGE = 320000
D_FEAT = 128

NC = 2
NS = 16
EPT = E_EDGE // NS
CH = 80
NCHUNK = EPT // CH
ROWS_PT = 624
TAIL_OFF = NS * ROWS_PT
TAIL = N_NODE - TAIL_OFF


def _sc_body(xu, xi, su, du, si, di, zeros, out_u_pre, out_i_pre,
             idx_s, idx_d, rows, acc, gsem):
    c = lax.axis_index("c")
    s = lax.axis_index("s")

    pltpu.sync_copy(zeros.at[pl.ds(s * ROWS_PT, ROWS_PT)],
                    acc.at[pl.ds(s * ROWS_PT, ROWS_PT)])

    @pl.when(s == NS - 1)
    def _():
        pltpu.sync_copy(zeros.at[pl.ds(TAIL_OFF, TAIL)],
                        acc.at[pl.ds(TAIL_OFF, TAIL)])

    plsc.subcore_barrier()

    def accumulate(x_hbm, src_hbm, dst_hbm):
        base0 = s * EPT

        @pl.loop(0, NCHUNK)
        def _chunk(i):
            base = base0 + i * CH
            pltpu.sync_copy(src_hbm.at[pl.ds(base, CH)], idx_s)
            pltpu.sync_copy(dst_hbm.at[pl.ds(base, CH)], idx_d)
            pltpu.async_copy(x_hbm.at[idx_s], rows, gsem).wait()
            pltpu.sync_copy(rows, acc.at[idx_d], add=True)

    @pl.when(c == 0)
    def _():
        accumulate(xu, su, du)

    @pl.when(c == 1)
    def _():
        accumulate(xi, si, di)

    plsc.subcore_barrier()

    def dump(out_ref):
        pltpu.sync_copy(acc.at[pl.ds(s * ROWS_PT, ROWS_PT)],
                        out_ref.at[pl.ds(s * ROWS_PT, ROWS_PT)])

        @pl.when(s == NS - 1)
        def _():
            pltpu.sync_copy(acc.at[pl.ds(TAIL_OFF, TAIL)],
                            out_ref.at[pl.ds(TAIL_OFF, TAIL)])

    @pl.when(c == 0)
    def _():
        dump(out_i_pre)

    @pl.when(c == 1)
    def _():
        dump(out_u_pre)


_sc_segment_sum = pl.kernel(
    _sc_body,
    out_type=(
        jax.ShapeDtypeStruct((N_NODE, D_FEAT), jnp.float32),
        jax.ShapeDtypeStruct((N_NODE, D_FEAT), jnp.float32),
    ),
    mesh=plsc.VectorSubcoreMesh(
        core_axis_name="c", subcore_axis_name="s",
        num_cores=NC, num_subcores=NS),
    scratch_types=[
        pltpu.VMEM((CH,), jnp.int32),
        pltpu.VMEM((CH,), jnp.int32),
        pltpu.VMEM((CH, D_FEAT), jnp.float32),
        pltpu.VMEM_SHARED((N_NODE, D_FEAT), jnp.float32),
        pltpu.SemaphoreType.DMA,
    ],
)


def _mm_body(pu_ref, pi_ref, wu_ref, wi_ref, ou_ref, oi_ref):
    ou_ref[...] = jnp.dot(pu_ref[...], wi_ref[...],
                          preferred_element_type=jnp.float32)
    oi_ref[...] = jnp.dot(pi_ref[...], wu_ref[...],
                          preferred_element_type=jnp.float32)


_MM_BLK = 1000


def _apply_weights(p_user, p_item, W_u2i, W_i2u):
    grid = (N_NODE // _MM_BLK,)
    blk = pl.BlockSpec((_MM_BLK, D_FEAT), lambda i: (i, 0))
    wblk = pl.BlockSpec((D_FEAT, D_FEAT), lambda i: (0, 0))
    return pl.pallas_call(
        _mm_body,
        grid=grid,
        in_specs=[blk, blk, wblk, wblk],
        out_specs=[blk, blk],
        out_shape=(
            jax.ShapeDtypeStruct((N_NODE, D_FEAT), jnp.float32),
            jax.ShapeDtypeStruct((N_NODE, D_FEAT), jnp.float32),
        ),
    )(p_user, p_item, W_u2i, W_i2u)


@jax.jit
def kernel(x_user, x_item, W_u2i, W_i2u,
           src_idx_u2i, dst_idx_u2i, src_idx_i2u, dst_idx_i2u):
    zeros = jnp.zeros((N_NODE, D_FEAT), jnp.float32)
    p_user, p_item = _sc_segment_sum(
        x_user, x_item, src_idx_u2i, dst_idx_u2i, src_idx_i2u, dst_idx_i2u,
        zeros)
    out_user, out_item = _apply_weights(p_user, p_item, W_u2i, W_i2u)
    return (out_user, out_item)


# speedup vs baseline: 12.3067x; 2.5186x over previous
import functools

import jax
import jax.numpy as jnp
from jax import lax
from jax.experimental import pallas as pl
from jax.experimental.pallas import tpu as pltpu
from jax.experimental.pallas import tpu_sc as plsc

N_NODE = 10000
E_EDGE = 320000
D_FEAT = 128

NC = 2
NS = 16
EPT = E_EDGE // NS
CH = 80
NCHUNK = EPT // CH
ROWS_PT = 624
TAIL_OFF = NS * ROWS_PT
TAIL = N_NODE - TAIL_OFF


def _sc_body(xu, xi, su, du, si, di, zeros, out_u_pre, out_i_pre,
             idx_s, idx_d0, idx_d1, rows0, rows1, acc,
             gsem0, gsem1, ssem0, ssem1, dsem0, dsem1):
    c = lax.axis_index("c")
    s = lax.axis_index("s")
    idx_d = (idx_d0, idx_d1)
    rows = (rows0, rows1)
    gsem = (gsem0, gsem1)
    ssem = (ssem0, ssem1)
    dsem = (dsem0, dsem1)

    pltpu.sync_copy(zeros.at[pl.ds(s * ROWS_PT, ROWS_PT)],
                    acc.at[pl.ds(s * ROWS_PT, ROWS_PT)])

    @pl.when(s == NS - 1)
    def _():
        pltpu.sync_copy(zeros.at[pl.ds(TAIL_OFF, TAIL)],
                        acc.at[pl.ds(TAIL_OFF, TAIL)])

    plsc.subcore_barrier()

    def accumulate(x_hbm, src_hbm, dst_hbm):
        base0 = s * EPT
        pltpu.sync_copy(src_hbm.at[pl.ds(base0, EPT)], idx_s)

        def start_gather(j, b):
            return pltpu.async_copy(
                x_hbm.at[idx_s.at[pl.ds(j * CH, CH)]], rows[b], gsem[b])

        def start_scatter(j, b):
            del j
            return pltpu.async_copy(rows[b], acc.at[idx_d[b]], ssem[b],
                                    add=True)

        def wait_gather(j, b):
            pltpu.make_async_copy(
                x_hbm.at[idx_s.at[pl.ds(j * CH, CH)]], rows[b],
                gsem[b]).wait()

        def wait_scatter(j, b):
            del j
            pltpu.make_async_copy(rows[b], acc.at[idx_d[b]], ssem[b]).wait()

        def start_dst_load(j, b):
            return pltpu.async_copy(dst_hbm.at[pl.ds(base0 + j * CH, CH)],
                                    idx_d[b], dsem[b])

        def wait_dst_load(j, b):
            pltpu.make_async_copy(dst_hbm.at[pl.ds(base0 + j * CH, CH)],
                                  idx_d[b], dsem[b]).wait()

        start_gather(0, 0)
        start_dst_load(0, 0)

        @pl.loop(0, NCHUNK, step=2)
        def _outer(i):
            @pl.when(i > 0)
            def _():
                wait_scatter(i - 1, 1)
            start_gather(i + 1, 1)
            start_dst_load(i + 1, 1)
            wait_gather(i, 0)
            wait_dst_load(i, 0)
            start_scatter(i, 0)

            wait_scatter(i, 0)

            @pl.when(i < NCHUNK - 2)
            def _():
                start_gather(i + 2, 0)
                start_dst_load(i + 2, 0)
            wait_gather(i + 1, 1)
            wait_dst_load(i + 1, 1)
            start_scatter(i + 1, 1)

        wait_scatter(NCHUNK - 1, 1)

    @pl.when(c == 0)
    def _():
        accumulate(xu, su, du)

    @pl.when(c == 1)
    def _():
        accumulate(xi, si, di)

    plsc.subcore_barrier()

    def dump(out_ref):
        pltpu.sync_copy(acc.at[pl.ds(s * ROWS_PT, ROWS_PT)],
                        out_ref.at[pl.ds(s * ROWS_PT, ROWS_PT)])

        @pl.when(s == NS - 1)
        def _():
            pltpu.sync_copy(acc.at[pl.ds(TAIL_OFF, TAIL)],
                            out_ref.at[pl.ds(TAIL_OFF, TAIL)])

    @pl.when(c == 0)
    def _():
        dump(out_i_pre)

    @pl.when(c == 1)
    def _():
        dump(out_u_pre)


_sc_segment_sum = pl.kernel(
    _sc_body,
    out_type=(
        jax.ShapeDtypeStruct((N_NODE, D_FEAT), jnp.float32),
        jax.ShapeDtypeStruct((N_NODE, D_FEAT), jnp.float32),
    ),
    mesh=plsc.VectorSubcoreMesh(
        core_axis_name="c", subcore_axis_name="s",
        num_cores=NC, num_subcores=NS),
    scratch_types=[
        pltpu.VMEM((EPT,), jnp.int32),
        pltpu.VMEM((CH,), jnp.int32),
        pltpu.VMEM((CH,), jnp.int32),
        pltpu.VMEM((CH, D_FEAT), jnp.float32),
        pltpu.VMEM((CH, D_FEAT), jnp.float32),
        pltpu.VMEM_SHARED((N_NODE, D_FEAT), jnp.float32),
        pltpu.SemaphoreType.DMA,
        pltpu.SemaphoreType.DMA,
        pltpu.SemaphoreType.DMA,
        pltpu.SemaphoreType.DMA,
        pltpu.SemaphoreType.DMA,
        pltpu.SemaphoreType.DMA,
    ],
)


def _mm_body(pu_ref, pi_ref, wu_ref, wi_ref, ou_ref, oi_ref):
    ou_ref[...] = jnp.dot(pu_ref[...], wi_ref[...],
                          preferred_element_type=jnp.float32)
    oi_ref[...] = jnp.dot(pi_ref[...], wu_ref[...],
                          preferred_element_type=jnp.float32)


_MM_BLK = 1000


def _apply_weights(p_user, p_item, W_u2i, W_i2u):
    grid = (N_NODE // _MM_BLK,)
    blk = pl.BlockSpec((_MM_BLK, D_FEAT), lambda i: (i, 0))
    wblk = pl.BlockSpec((D_FEAT, D_FEAT), lambda i: (0, 0))
    return pl.pallas_call(
        _mm_body,
        grid=grid,
        in_specs=[blk, blk, wblk, wblk],
        out_specs=[blk, blk],
        out_shape=(
            jax.ShapeDtypeStruct((N_NODE, D_FEAT), jnp.float32),
            jax.ShapeDtypeStruct((N_NODE, D_FEAT), jnp.float32),
        ),
    )(p_user, p_item, W_u2i, W_i2u)


@jax.jit
def kernel(x_user, x_item, W_u2i, W_i2u,
           src_idx_u2i, dst_idx_u2i, src_idx_i2u, dst_idx_i2u):
    zeros = jnp.zeros((N_NODE, D_FEAT), jnp.float32)
    p_user, p_item = _sc_segment_sum(
        x_user, x_item, src_idx_u2i, dst_idx_u2i, src_idx_i2u, dst_idx_i2u,
        zeros)
    out_user, out_item = _apply_weights(p_user, p_item, W_u2i, W_i2u)
    return (out_user, out_item)
